# baseline (speedup 1.0000x reference)
import functools

import jax
import jax.numpy as jnp
from jax import lax
from jax.experimental import pallas as pl
from jax.experimental.pallas import tpu as pltpu
from jax.experimental.pallas import tpu_sc as plsc

_LANES = 16
_BLANES = 32
_NC = 2
_NS = 16
_NBUF = 5


def _mm_body(x_ref, w_ref, o_ref):
    o_ref[...] = jnp.dot(
        x_ref[...], w_ref[...], preferred_element_type=jnp.float32
    ).astype(jnp.bfloat16)


@functools.partial(jax.jit, static_argnames=("block_rows",))
def _matmul(x, w, block_rows=1000):
    n, d_in = x.shape
    d_out = w.shape[1]
    nb = n // block_rows
    return pl.pallas_call(
        _mm_body,
        grid=(nb,),
        in_specs=[
            pl.BlockSpec((block_rows, d_in), lambda i: (i, 0)),
            pl.BlockSpec((d_in, d_out), lambda i: (0, 0)),
        ],
        out_specs=pl.BlockSpec((block_rows, d_out), lambda i: (i, 0)),
        out_shape=jax.ShapeDtypeStruct((n, d_out), jnp.bfloat16),
    )(x, w)


def _comb_body(p_ref, b_ref, o_ref):
    o_ref[...] = (p_ref[0].astype(jnp.float32) + p_ref[1].astype(jnp.float32)
                  + b_ref[0][None, :])


@functools.partial(jax.jit, static_argnames=("block_rows",))
def _combine(parts, bias, block_rows=1000):
    _, n, d_out = parts.shape
    nb = n // block_rows
    return pl.pallas_call(
        _comb_body,
        grid=(nb,),
        in_specs=[
            pl.BlockSpec((_NC, block_rows, d_out), lambda i: (0, i, 0)),
            pl.BlockSpec((1, d_out), lambda i: (0, 0)),
        ],
        out_specs=pl.BlockSpec((block_rows, d_out), lambda i: (i, 0)),
        out_shape=jax.ShapeDtypeStruct((n, d_out), jnp.float32),
    )(parts, bias.reshape(1, d_out))


def _make_sc_agg(n_nodes, n_edges, d, chunk):
    nw = _NC * _NS
    epw = n_edges // nw
    nch = epw // chunk
    rps = n_nodes // _NS
    zrows = 25
    nz = rps // zrows
    nbvec = d // _BLANES

    mesh = plsc.VectorSubcoreMesh(
        core_axis_name="c", subcore_axis_name="s",
        num_cores=_NC, num_subcores=_NS)

    @functools.partial(
        pl.kernel,
        out_type=jax.ShapeDtypeStruct((_NC, n_nodes, d), jnp.bfloat16),
        mesh=mesh,
        compiler_params=pltpu.CompilerParams(
            use_tc_tiling_on_sc=False, needs_layout_passes=False),
        scratch_types=[
            pltpu.VMEM_SHARED((n_nodes, d), jnp.bfloat16),
            pltpu.VMEM((nch, chunk), jnp.int32),
            pltpu.VMEM((nch, chunk), jnp.int32),
            pltpu.VMEM((epw,), jnp.float32),
            [pltpu.VMEM((chunk, d), jnp.bfloat16)] * _NBUF,
            pltpu.VMEM((zrows, d), jnp.bfloat16),
            [pltpu.SemaphoreType.DMA] * _NBUF,
            [pltpu.SemaphoreType.DMA] * _NBUF,
        ],
    )
    def sc_agg(src_hbm, dst_hbm, ew_hbm, sup_hbm, out_hbm,
               acc, srcv, dstv, wv, rows_ring, zbuf, gsems, ssems):
        cid = lax.axis_index("c")
        sid = lax.axis_index("s")
        wid = cid * _NS + sid

        pltpu.sync_copy(src_hbm.at[wid], srcv)
        pltpu.sync_copy(dst_hbm.at[wid], dstv)
        pltpu.sync_copy(ew_hbm.at[wid], wv)

        zero = jnp.zeros((_BLANES,), jnp.bfloat16)

        def fillz(r, _):
            for dd in range(nbvec):
                zbuf[r, pl.ds(dd * _BLANES, _BLANES)] = zero
            return 0

        lax.fori_loop(0, zrows, fillz, 0)
        for p in range(nz):
            pltpu.sync_copy(zbuf, acc.at[pl.ds(sid * rps + p * zrows, zrows)])
        plsc.subcore_barrier()

        def start_gather(j, r):
            pltpu.async_copy(sup_hbm.at[srcv.at[j]], rows_ring[r], gsems[r])

        def wait_gather(j, r):
            pltpu.make_async_copy(
                sup_hbm.at[srcv.at[j]], rows_ring[r], gsems[r]).wait()

        def start_scatter(j, r):
            pltpu.async_copy(rows_ring[r], acc.at[dstv.at[j]], ssems[r],
                             add=True)

        def wait_scatter(j, r):
            pltpu.make_async_copy(
                rows_ring[r], acc.at[dstv.at[j]], ssems[r]).wait()

        def scale(j, r):
            rows = rows_ring[r]

            def body(k, _):
                wb = plsc.load_gather(
                    wv, [jnp.full((_LANES,), j * chunk + k, jnp.int32)])
                wb2 = plsc.pack(wb, wb, format=plsc.PackFormat.INTERLEAVED)
                for dd in range(nbvec):
                    sl = pl.ds(dd * _BLANES, _BLANES)
                    rows[k, sl] = rows[k, sl] * wb2
                return 0

            lax.fori_loop(0, chunk, body, 0, unroll=5)

        for r in range(4):
            start_gather(jnp.int32(r), r)

        def pipe_body(t, _):
            for r in range(_NBUF):
                j = t * _NBUF + r
                wait_gather(j, r)
                scale(j, r)
                start_scatter(j, r)
                rq = (r - 1) % _NBUF

                @pl.when(j >= 1)
                def _():
                    wait_scatter(j - 1, rq)

                @pl.when(j + 4 < nch)
                def _():
                    start_gather(j + 4, rq)
            return 0

        lax.fori_loop(0, nch // _NBUF, pipe_body, 0)
        wait_scatter(nch - 1, (nch - 1) % _NBUF)
        plsc.subcore_barrier()

        for p in range(nz):
            r0 = sid * rps + p * zrows
            pltpu.sync_copy(acc.at[pl.ds(r0, zrows)], zbuf)
            pltpu.sync_copy(zbuf, out_hbm.at[cid, pl.ds(r0, zrows)])

    return sc_agg


_CHUNK = 125


@functools.cache
def _get_sc_agg(n_nodes, n_edges, d):
    return _make_sc_agg(n_nodes, n_edges, d, _CHUNK)


def kernel(edge_index, edge_weight, in_feature, weight, bias):
    n, _ = in_feature.shape
    d_out = weight.shape[1]
    e = edge_weight.shape[0]
    nw = _NC * _NS
    nch = (e // nw) // _CHUNK

    support = _matmul(in_feature, weight)

    src = edge_index[1].astype(jnp.int32).reshape(nw, nch, _CHUNK)
    dst = edge_index[0].astype(jnp.int32).reshape(nw, nch, _CHUNK)
    ew = edge_weight.reshape(nw, e // nw)

    sc_agg = _get_sc_agg(n, e, d_out)
    parts = sc_agg(src, dst, ew, support)
    return _combine(parts, bias)


# speedup vs baseline: 1.0009x; 1.0009x over previous
import functools

import jax
import jax.numpy as jnp
from jax import lax
from jax.experimental import pallas as pl
from jax.experimental.pallas import tpu as pltpu
from jax.experimental.pallas import tpu_sc as plsc

_LANES = 16
_BLANES = 32
_NC = 2
_NS = 16
_NBUF = 5


def _mm_body(x_ref, w_ref, o_ref):
    o_ref[...] = jnp.dot(
        x_ref[...], w_ref[...], preferred_element_type=jnp.float32
    ).astype(jnp.bfloat16)


@functools.partial(jax.jit, static_argnames=("block_rows",))
def _matmul(x, w, block_rows=1000):
    n, d_in = x.shape
    d_out = w.shape[1]
    nb = n // block_rows
    return pl.pallas_call(
        _mm_body,
        grid=(nb,),
        in_specs=[
            pl.BlockSpec((block_rows, d_in), lambda i: (i, 0)),
            pl.BlockSpec((d_in, d_out), lambda i: (0, 0)),
        ],
        out_specs=pl.BlockSpec((block_rows, d_out), lambda i: (i, 0)),
        out_shape=jax.ShapeDtypeStruct((n, d_out), jnp.bfloat16),
    )(x, w)


def _comb_body(p_ref, b_ref, o_ref):
    o_ref[...] = (p_ref[0].astype(jnp.float32) + p_ref[1].astype(jnp.float32)
                  + b_ref[0][None, :])


@functools.partial(jax.jit, static_argnames=("block_rows",))
def _combine(parts, bias, block_rows=1000):
    _, n, d_out = parts.shape
    nb = n // block_rows
    return pl.pallas_call(
        _comb_body,
        grid=(nb,),
        in_specs=[
            pl.BlockSpec((_NC, block_rows, d_out), lambda i: (0, i, 0)),
            pl.BlockSpec((1, d_out), lambda i: (0, 0)),
        ],
        out_specs=pl.BlockSpec((block_rows, d_out), lambda i: (i, 0)),
        out_shape=jax.ShapeDtypeStruct((n, d_out), jnp.float32),
    )(parts, bias.reshape(1, d_out))


def _make_sc_agg(n_nodes, n_edges, d, chunk):
    nw = _NC * _NS
    epw = n_edges // nw
    nch = epw // chunk
    rps = n_nodes // _NS
    zrows = 25
    nz = rps // zrows
    nbvec = d // _BLANES

    mesh = plsc.VectorSubcoreMesh(
        core_axis_name="c", subcore_axis_name="s",
        num_cores=_NC, num_subcores=_NS)

    @functools.partial(
        pl.kernel,
        out_type=jax.ShapeDtypeStruct((_NC, n_nodes, d), jnp.bfloat16),
        mesh=mesh,
        compiler_params=pltpu.CompilerParams(
            use_tc_tiling_on_sc=False, needs_layout_passes=False),
        scratch_types=[
            pltpu.VMEM_SHARED((n_nodes, d), jnp.bfloat16),
            pltpu.VMEM((nch, chunk), jnp.int32),
            pltpu.VMEM((nch, chunk), jnp.int32),
            pltpu.VMEM((epw,), jnp.float32),
            [pltpu.VMEM((chunk, d), jnp.bfloat16)] * _NBUF,
            pltpu.VMEM((zrows, d), jnp.bfloat16),
            [pltpu.SemaphoreType.DMA] * _NBUF,
            [pltpu.SemaphoreType.DMA] * _NBUF,
        ],
    )
    def sc_agg(src_hbm, dst_hbm, ew_hbm, sup_hbm, out_hbm,
               acc, srcv, dstv, wv, rows_ring, zbuf, gsems, ssems):
        cid = lax.axis_index("c")
        sid = lax.axis_index("s")
        wid = cid * _NS + sid

        pltpu.sync_copy(src_hbm.at[wid], srcv)
        pltpu.sync_copy(dst_hbm.at[wid], dstv)
        pltpu.sync_copy(ew_hbm.at[wid], wv)

        zero = jnp.zeros((_BLANES,), jnp.bfloat16)

        def fillz(r, _):
            for dd in range(nbvec):
                zbuf[r, pl.ds(dd * _BLANES, _BLANES)] = zero
            return 0

        lax.fori_loop(0, zrows, fillz, 0)
        for p in range(nz):
            pltpu.sync_copy(zbuf, acc.at[pl.ds(sid * rps + p * zrows, zrows)])
        plsc.subcore_barrier()

        def start_gather(j, r):
            pltpu.async_copy(sup_hbm.at[srcv.at[j]], rows_ring[r], gsems[r])

        def wait_gather(j, r):
            pltpu.make_async_copy(
                sup_hbm.at[srcv.at[j]], rows_ring[r], gsems[r]).wait()

        def start_scatter(j, r):
            pltpu.async_copy(rows_ring[r], acc.at[dstv.at[j]], ssems[r],
                             add=True)

        def wait_scatter(j, r):
            pltpu.make_async_copy(
                rows_ring[r], acc.at[dstv.at[j]], ssems[r]).wait()

        def scale(j, r):
            rows = rows_ring[r]

            def body(k, _):
                wb = plsc.load_gather(
                    wv, [jnp.full((_LANES,), j * chunk + k, jnp.int32)])
                wb2 = plsc.pack(wb, wb, format=plsc.PackFormat.INTERLEAVED)
                for dd in range(nbvec):
                    sl = pl.ds(dd * _BLANES, _BLANES)
                    rows[k, sl] = rows[k, sl] * wb2
                return 0

            lax.fori_loop(0, chunk, body, 0, unroll=5)

        for r in range(3):
            start_gather(jnp.int32(r), r)

        def pipe_body(t, _):
            for r in range(_NBUF):
                j = t * _NBUF + r
                wait_gather(j, r)
                scale(j, r)
                start_scatter(j, r)
                rq = (r - 2) % _NBUF

                @pl.when(j >= 2)
                def _():
                    wait_scatter(j - 2, rq)

                @pl.when(j + 3 < nch)
                def _():
                    start_gather(j + 3, rq)
            return 0

        lax.fori_loop(0, nch // _NBUF, pipe_body, 0)
        wait_scatter(nch - 2, (nch - 2) % _NBUF)
        wait_scatter(nch - 1, (nch - 1) % _NBUF)
        plsc.subcore_barrier()

        for p in range(nz):
            r0 = sid * rps + p * zrows
            pltpu.sync_copy(acc.at[pl.ds(r0, zrows)], zbuf)
            pltpu.sync_copy(zbuf, out_hbm.at[cid, pl.ds(r0, zrows)])

    return sc_agg


_CHUNK = 125


@functools.cache
def _get_sc_agg(n_nodes, n_edges, d):
    return _make_sc_agg(n_nodes, n_edges, d, _CHUNK)


def kernel(edge_index, edge_weight, in_feature, weight, bias):
    n, _ = in_feature.shape
    d_out = weight.shape[1]
    e = edge_weight.shape[0]
    nw = _NC * _NS
    nch = (e // nw) // _CHUNK

    support = _matmul(in_feature, weight)

    src = edge_index[1].astype(jnp.int32).reshape(nw, nch, _CHUNK)
    dst = edge_index[0].astype(jnp.int32).reshape(nw, nch, _CHUNK)
    ew = edge_weight.reshape(nw, e // nw)

    sc_agg = _get_sc_agg(n, e, d_out)
    parts = sc_agg(src, dst, ew, support)
    return _combine(parts, bias)


# speedup vs baseline: 1.0255x; 1.0246x over previous
import functools

import jax
import jax.numpy as jnp
from jax import lax
from jax.experimental import pallas as pl
from jax.experimental.pallas import tpu as pltpu
from jax.experimental.pallas import tpu_sc as plsc

_LANES = 16
_BLANES = 32
_NC = 2
_NS = 16
_NBUF = 5


def _mm_body(x_ref, w_ref, o_ref):
    o_ref[...] = jnp.dot(
        x_ref[...], w_ref[...], preferred_element_type=jnp.float32
    ).astype(jnp.bfloat16)


@functools.partial(jax.jit, static_argnames=("block_rows",))
def _matmul(x, w, block_rows=2000):
    n, d_in = x.shape
    d_out = w.shape[1]
    nb = n // block_rows
    return pl.pallas_call(
        _mm_body,
        grid=(nb,),
        in_specs=[
            pl.BlockSpec((block_rows, d_in), lambda i: (i, 0)),
            pl.BlockSpec((d_in, d_out), lambda i: (0, 0)),
        ],
        out_specs=pl.BlockSpec((block_rows, d_out), lambda i: (i, 0)),
        out_shape=jax.ShapeDtypeStruct((n, d_out), jnp.bfloat16),
    )(x, w)


def _comb_body(p_ref, b_ref, o_ref):
    o_ref[...] = (p_ref[0].astype(jnp.float32) + p_ref[1].astype(jnp.float32)
                  + b_ref[0][None, :])


@functools.partial(jax.jit, static_argnames=("block_rows",))
def _combine(parts, bias, block_rows=2000):
    _, n, d_out = parts.shape
    nb = n // block_rows
    return pl.pallas_call(
        _comb_body,
        grid=(nb,),
        in_specs=[
            pl.BlockSpec((_NC, block_rows, d_out), lambda i: (0, i, 0)),
            pl.BlockSpec((1, d_out), lambda i: (0, 0)),
        ],
        out_specs=pl.BlockSpec((block_rows, d_out), lambda i: (i, 0)),
        out_shape=jax.ShapeDtypeStruct((n, d_out), jnp.float32),
    )(parts, bias.reshape(1, d_out))


def _make_sc_agg(n_nodes, n_edges, d, chunk):
    nw = _NC * _NS
    epw = n_edges // nw
    nch = epw // chunk
    rps = n_nodes // _NS
    zrows = 25
    nz = rps // zrows
    nbvec = d // _BLANES

    mesh = plsc.VectorSubcoreMesh(
        core_axis_name="c", subcore_axis_name="s",
        num_cores=_NC, num_subcores=_NS)

    @functools.partial(
        pl.kernel,
        out_type=jax.ShapeDtypeStruct((_NC, n_nodes, d), jnp.bfloat16),
        mesh=mesh,
        compiler_params=pltpu.CompilerParams(
            use_tc_tiling_on_sc=False, needs_layout_passes=False),
        scratch_types=[
            pltpu.VMEM_SHARED((n_nodes, d), jnp.bfloat16),
            pltpu.VMEM((nch, chunk), jnp.int32),
            pltpu.VMEM((nch, chunk), jnp.int32),
            pltpu.VMEM((epw,), jnp.float32),
            [pltpu.VMEM((chunk, d), jnp.bfloat16)] * _NBUF,
            pltpu.VMEM((zrows, d), jnp.bfloat16),
            [pltpu.SemaphoreType.DMA] * _NBUF,
            [pltpu.SemaphoreType.DMA] * _NBUF,
        ],
    )
    def sc_agg(src_hbm, dst_hbm, ew_hbm, sup_hbm, out_hbm,
               acc, srcv, dstv, wv, rows_ring, zbuf, gsems, ssems):
        cid = lax.axis_index("c")
        sid = lax.axis_index("s")
        wid = cid * _NS + sid

        pltpu.sync_copy(src_hbm.at[wid], srcv)
        pltpu.sync_copy(dst_hbm.at[wid], dstv)
        pltpu.sync_copy(ew_hbm.at[wid], wv)

        zero = jnp.zeros((_BLANES,), jnp.bfloat16)

        def fillz(r, _):
            for dd in range(nbvec):
                zbuf[r, pl.ds(dd * _BLANES, _BLANES)] = zero
            return 0

        lax.fori_loop(0, zrows, fillz, 0)
        for p in range(nz):
            pltpu.sync_copy(zbuf, acc.at[pl.ds(sid * rps + p * zrows, zrows)])
        plsc.subcore_barrier()

        def start_gather(j, r):
            pltpu.async_copy(sup_hbm.at[srcv.at[j]], rows_ring[r], gsems[r])

        def wait_gather(j, r):
            pltpu.make_async_copy(
                sup_hbm.at[srcv.at[j]], rows_ring[r], gsems[r]).wait()

        def start_scatter(j, r):
            pltpu.async_copy(rows_ring[r], acc.at[dstv.at[j]], ssems[r],
                             add=True)

        def wait_scatter(j, r):
            pltpu.make_async_copy(
                rows_ring[r], acc.at[dstv.at[j]], ssems[r]).wait()

        def scale(j, r):
            rows = rows_ring[r]

            def body(k, _):
                wb = plsc.load_gather(
                    wv, [jnp.full((_LANES,), j * chunk + k, jnp.int32)])
                wb2 = plsc.pack(wb, wb, format=plsc.PackFormat.INTERLEAVED)
                for dd in range(nbvec):
                    sl = pl.ds(dd * _BLANES, _BLANES)
                    rows[k, sl] = rows[k, sl] * wb2
                return 0

            lax.fori_loop(0, chunk, body, 0, unroll=5)

        for r in range(3):
            start_gather(jnp.int32(r), r)

        def pipe_body(t, _):
            for r in range(_NBUF):
                j = t * _NBUF + r
                wait_gather(j, r)
                scale(j, r)
                start_scatter(j, r)
                rq = (r - 2) % _NBUF

                @pl.when(j >= 2)
                def _():
                    wait_scatter(j - 2, rq)

                @pl.when(j + 3 < nch)
                def _():
                    start_gather(j + 3, rq)
            return 0

        lax.fori_loop(0, nch // _NBUF, pipe_body, 0)
        wait_scatter(nch - 2, (nch - 2) % _NBUF)
        wait_scatter(nch - 1, (nch - 1) % _NBUF)
        plsc.subcore_barrier()

        for p in range(nz):
            r0 = sid * rps + p * zrows
            pltpu.sync_copy(acc.at[pl.ds(r0, zrows)], zbuf)
            pltpu.sync_copy(zbuf, out_hbm.at[cid, pl.ds(r0, zrows)])

    return sc_agg


_CHUNK = 125


@functools.cache
def _get_sc_agg(n_nodes, n_edges, d):
    return _make_sc_agg(n_nodes, n_edges, d, _CHUNK)


def kernel(edge_index, edge_weight, in_feature, weight, bias):
    n, _ = in_feature.shape
    d_out = weight.shape[1]
    e = edge_weight.shape[0]
    nw = _NC * _NS
    nch = (e // nw) // _CHUNK

    support = _matmul(in_feature, weight)

    src = edge_index[1].astype(jnp.int32).reshape(nw, nch, _CHUNK)
    dst = edge_index[0].astype(jnp.int32).reshape(nw, nch, _CHUNK)
    ew = edge_weight.reshape(nw, e // nw)

    sc_agg = _get_sc_agg(n, e, d_out)
    parts = sc_agg(src, dst, ew, support)
    return _combine(parts, bias)
